# baseline (speedup 1.0000x reference)
import functools
import math

import jax
import jax.numpy as jnp
from jax import lax
from jax.experimental import pallas as pl
from jax.experimental.pallas import tpu as pltpu
from jax.experimental.pallas import tpu_sc as plsc

H_DIM = 512
HALF = H_DIM // 2
BATCH = 16384
GAMMA = 12.0
EPSILON = 2.0
EMB_RANGE = (GAMMA + EPSILON) / H_DIM
PI = 3.141592653589793

NW = 32
CH = 64
NCHUNK = 8


def _sc_gather(embs, w_relation, h_idx, r_idx, t_idx, chunk_base, bc):
    mesh = plsc.VectorSubcoreMesh(core_axis_name="c", subcore_axis_name="s")
    bpw = bc // NW
    nsub = bpw // CH
    assert nsub in (1, 2)

    @functools.partial(
        pl.kernel,
        mesh=mesh,
        out_type=(
            jax.ShapeDtypeStruct((bc, H_DIM), jnp.float32),
            jax.ShapeDtypeStruct((bc, H_DIM), jnp.float32),
            jax.ShapeDtypeStruct((bc, HALF), jnp.float32),
        ),
        scratch_types=[
            pltpu.VMEM((bpw,), jnp.int32),
            pltpu.VMEM((bpw,), jnp.int32),
            pltpu.VMEM((bpw,), jnp.int32),
            pltpu.VMEM((CH, H_DIM), jnp.float32),
            pltpu.VMEM((CH, H_DIM), jnp.float32),
            pltpu.VMEM((CH, H_DIM), jnp.float32),
            pltpu.VMEM((CH, HALF), jnp.float32),
            pltpu.SemaphoreType.DMA,
            pltpu.SemaphoreType.DMA,
            pltpu.SemaphoreType.DMA,
            pltpu.SemaphoreType.DMA,
            pltpu.SemaphoreType.DMA,
            pltpu.SemaphoreType.DMA,
            pltpu.SemaphoreType.DMA,
            pltpu.SemaphoreType.DMA,
        ],
    )
    def k(embs_h, wrel_h, hidx_h, ridx_h, tidx_h,
          hout, tout, rout,
          hidx_v, ridx_v, tidx_v, b0, b1, b2, rb,
          gb0, gb1, gb2, grb, sb0, sb1, sb2, srb):
        wid = lax.axis_index("s") * 2 + lax.axis_index("c")
        base = wid * bpw
        src = chunk_base + base

        def sub(ref, j):
            return ref.at[pl.ds(j * CH, CH)]

        def out_sub(ref, j):
            return ref.at[pl.ds(base + j * CH, CH)]

        if nsub == 2:
            pltpu.sync_copy(hidx_h.at[pl.ds(src, bpw)], hidx_v)
            gh0 = pltpu.async_copy(embs_h.at[sub(hidx_v, 0)], b0, gb0)
            gh1 = pltpu.async_copy(embs_h.at[sub(hidx_v, 1)], b1, gb1)
            pltpu.sync_copy(tidx_h.at[pl.ds(src, bpw)], tidx_v)
            gt0 = pltpu.async_copy(embs_h.at[sub(tidx_v, 0)], b2, gb2)
            pltpu.sync_copy(ridx_h.at[pl.ds(src, bpw)], ridx_v)
            gr0 = pltpu.async_copy(wrel_h.at[sub(ridx_v, 0)], rb, grb)

            gh0.wait()
            sh0 = pltpu.async_copy(b0, out_sub(hout, 0), sb0)
            gh1.wait()
            sh1 = pltpu.async_copy(b1, out_sub(hout, 1), sb1)
            sh0.wait()
            gt1 = pltpu.async_copy(embs_h.at[sub(tidx_v, 1)], b0, gb0)
            gr0.wait()
            sr0 = pltpu.async_copy(rb, out_sub(rout, 0), srb)
            sr0.wait()
            gr1 = pltpu.async_copy(wrel_h.at[sub(ridx_v, 1)], rb, grb)
            gt0.wait()
            st0 = pltpu.async_copy(b2, out_sub(tout, 0), sb2)
            gt1.wait()
            st1 = pltpu.async_copy(b0, out_sub(tout, 1), sb0)
            gr1.wait()
            sr1 = pltpu.async_copy(rb, out_sub(rout, 1), srb)
            sh1.wait()
            st0.wait()
            st1.wait()
            sr1.wait()
        else:
            pltpu.sync_copy(hidx_h.at[pl.ds(src, bpw)], hidx_v)
            gh0 = pltpu.async_copy(embs_h.at[sub(hidx_v, 0)], b0, gb0)
            pltpu.sync_copy(tidx_h.at[pl.ds(src, bpw)], tidx_v)
            gt0 = pltpu.async_copy(embs_h.at[sub(tidx_v, 0)], b1, gb1)
            pltpu.sync_copy(ridx_h.at[pl.ds(src, bpw)], ridx_v)
            gr0 = pltpu.async_copy(wrel_h.at[sub(ridx_v, 0)], rb, grb)
            gh0.wait()
            sh0 = pltpu.async_copy(b0, out_sub(hout, 0), sb0)
            gt0.wait()
            st0 = pltpu.async_copy(b1, out_sub(tout, 0), sb1)
            gr0.wait()
            sr0 = pltpu.async_copy(rb, out_sub(rout, 0), srb)
            sh0.wait()
            st0.wait()
            sr0.wait()

    return k(embs, w_relation, h_idx, r_idx, t_idx)


_SIN_C = (9.999999992634e-01, -1.666666592737e-01, 8.333321297382e-03,
          -1.984053414314e-04, 2.753585048001e-06, -2.472881380150e-08,
          1.361309747309e-10)
_COS_C = (1.000000000293e+00, -4.999999985941e-01, 4.166666351410e-02,
          -1.388886311125e-03, 2.480055413054e-05, -2.753480385845e-07,
          2.060360183243e-09, -9.722486996111e-12)


def _horner(u, coeffs):
    acc = jnp.full_like(u, coeffs[-1])
    for c in coeffs[-2::-1]:
        acc = acc * u + c
    return acc


def _tc_score(head, tail, rel):
    b = head.shape[0]
    BR = min(1024, b)
    scale = EMB_RANGE / math.sqrt(3.0)
    inv_phase = PI / EMB_RANGE

    def body(h_ref, t_ref, r_ref, o_ref):
        h = h_ref[...]
        t = t_ref[...]
        r = r_ref[...]
        re_h = h[:, :HALF] * scale
        im_h = h[:, HALF:] * scale
        phase = r * inv_phase
        u = phase * phase
        cr = _horner(u, _COS_C)
        sr = _horner(u, _SIN_C) * phase
        re_s = re_h * cr - im_h * sr - t[:, :HALF] * scale
        im_s = re_h * sr + im_h * cr - t[:, HALF:] * scale
        dist = jnp.sqrt(re_s * re_s + im_s * im_s)
        o_ref[...] = GAMMA - jnp.sum(dist, axis=1, keepdims=True)

    return pl.pallas_call(
        body,
        grid=(b // BR,),
        in_specs=[
            pl.BlockSpec((BR, H_DIM), lambda i: (i, 0)),
            pl.BlockSpec((BR, H_DIM), lambda i: (i, 0)),
            pl.BlockSpec((BR, HALF), lambda i: (i, 0)),
        ],
        out_specs=pl.BlockSpec((BR, 1), lambda i: (i, 0)),
        out_shape=jax.ShapeDtypeStruct((b, 1), jnp.float32),
    )(head, tail, rel)


def kernel(embs, sample, w_relation):
    h_idx = sample[0]
    r_idx = sample[1]
    t_idx = sample[2]
    bc = BATCH // NCHUNK
    scores = []
    for i in range(NCHUNK):
        head, tail, rel = _sc_gather(
            embs, w_relation, h_idx, r_idx, t_idx, i * bc, bc)
        scores.append(_tc_score(head, tail, rel))
    return jnp.concatenate(scores, axis=0)


# speedup vs baseline: 1.1120x; 1.1120x over previous
import functools
import math

import jax
import jax.numpy as jnp
from jax import lax
from jax.experimental import pallas as pl
from jax.experimental.pallas import tpu as pltpu
from jax.experimental.pallas import tpu_sc as plsc

H_DIM = 512
HALF = H_DIM // 2
BATCH = 16384
GAMMA = 12.0
EPSILON = 2.0
EMB_RANGE = (GAMMA + EPSILON) / H_DIM
PI = 3.141592653589793

NW = 32
CH = 64
NCHUNK = 4


def _sc_gather(embs, w_relation, h_idx, r_idx, t_idx, chunk_base, bc):
    mesh = plsc.VectorSubcoreMesh(core_axis_name="c", subcore_axis_name="s")
    bpw = bc // NW
    nsub = bpw // CH
    assert nsub in (1, 2)

    @functools.partial(
        pl.kernel,
        mesh=mesh,
        out_type=(
            jax.ShapeDtypeStruct((bc, H_DIM), jnp.float32),
            jax.ShapeDtypeStruct((bc, H_DIM), jnp.float32),
            jax.ShapeDtypeStruct((bc, HALF), jnp.float32),
        ),
        scratch_types=[
            pltpu.VMEM((bpw,), jnp.int32),
            pltpu.VMEM((bpw,), jnp.int32),
            pltpu.VMEM((bpw,), jnp.int32),
            pltpu.VMEM((CH, H_DIM), jnp.float32),
            pltpu.VMEM((CH, H_DIM), jnp.float32),
            pltpu.VMEM((CH, H_DIM), jnp.float32),
            pltpu.VMEM((CH, HALF), jnp.float32),
            pltpu.SemaphoreType.DMA,
            pltpu.SemaphoreType.DMA,
            pltpu.SemaphoreType.DMA,
            pltpu.SemaphoreType.DMA,
            pltpu.SemaphoreType.DMA,
            pltpu.SemaphoreType.DMA,
            pltpu.SemaphoreType.DMA,
            pltpu.SemaphoreType.DMA,
        ],
    )
    def k(embs_h, wrel_h, hidx_h, ridx_h, tidx_h,
          hout, tout, rout,
          hidx_v, ridx_v, tidx_v, b0, b1, b2, rb,
          gb0, gb1, gb2, grb, sb0, sb1, sb2, srb):
        wid = lax.axis_index("s") * 2 + lax.axis_index("c")
        base = wid * bpw
        src = chunk_base + base

        def sub(ref, j):
            return ref.at[pl.ds(j * CH, CH)]

        def out_sub(ref, j):
            return ref.at[pl.ds(base + j * CH, CH)]

        if nsub == 2:
            pltpu.sync_copy(hidx_h.at[pl.ds(src, bpw)], hidx_v)
            gh0 = pltpu.async_copy(embs_h.at[sub(hidx_v, 0)], b0, gb0)
            gh1 = pltpu.async_copy(embs_h.at[sub(hidx_v, 1)], b1, gb1)
            pltpu.sync_copy(tidx_h.at[pl.ds(src, bpw)], tidx_v)
            gt0 = pltpu.async_copy(embs_h.at[sub(tidx_v, 0)], b2, gb2)
            pltpu.sync_copy(ridx_h.at[pl.ds(src, bpw)], ridx_v)
            gr0 = pltpu.async_copy(wrel_h.at[sub(ridx_v, 0)], rb, grb)

            gh0.wait()
            sh0 = pltpu.async_copy(b0, out_sub(hout, 0), sb0)
            gh1.wait()
            sh1 = pltpu.async_copy(b1, out_sub(hout, 1), sb1)
            sh0.wait()
            gt1 = pltpu.async_copy(embs_h.at[sub(tidx_v, 1)], b0, gb0)
            gr0.wait()
            sr0 = pltpu.async_copy(rb, out_sub(rout, 0), srb)
            sr0.wait()
            gr1 = pltpu.async_copy(wrel_h.at[sub(ridx_v, 1)], rb, grb)
            gt0.wait()
            st0 = pltpu.async_copy(b2, out_sub(tout, 0), sb2)
            gt1.wait()
            st1 = pltpu.async_copy(b0, out_sub(tout, 1), sb0)
            gr1.wait()
            sr1 = pltpu.async_copy(rb, out_sub(rout, 1), srb)
            sh1.wait()
            st0.wait()
            st1.wait()
            sr1.wait()
        else:
            pltpu.sync_copy(hidx_h.at[pl.ds(src, bpw)], hidx_v)
            gh0 = pltpu.async_copy(embs_h.at[sub(hidx_v, 0)], b0, gb0)
            pltpu.sync_copy(tidx_h.at[pl.ds(src, bpw)], tidx_v)
            gt0 = pltpu.async_copy(embs_h.at[sub(tidx_v, 0)], b1, gb1)
            pltpu.sync_copy(ridx_h.at[pl.ds(src, bpw)], ridx_v)
            gr0 = pltpu.async_copy(wrel_h.at[sub(ridx_v, 0)], rb, grb)
            gh0.wait()
            sh0 = pltpu.async_copy(b0, out_sub(hout, 0), sb0)
            gt0.wait()
            st0 = pltpu.async_copy(b1, out_sub(tout, 0), sb1)
            gr0.wait()
            sr0 = pltpu.async_copy(rb, out_sub(rout, 0), srb)
            sh0.wait()
            st0.wait()
            sr0.wait()

    return k(embs, w_relation, h_idx, r_idx, t_idx)


_SIN_C = (9.999999992634e-01, -1.666666592737e-01, 8.333321297382e-03,
          -1.984053414314e-04, 2.753585048001e-06, -2.472881380150e-08,
          1.361309747309e-10)
_COS_C = (1.000000000293e+00, -4.999999985941e-01, 4.166666351410e-02,
          -1.388886311125e-03, 2.480055413054e-05, -2.753480385845e-07,
          2.060360183243e-09, -9.722486996111e-12)


def _horner(u, coeffs):
    acc = jnp.full_like(u, coeffs[-1])
    for c in coeffs[-2::-1]:
        acc = acc * u + c
    return acc


def _tc_score(head, tail, rel):
    b = head.shape[0]
    BR = min(1024, b)
    scale = EMB_RANGE / math.sqrt(3.0)
    inv_phase = PI / EMB_RANGE

    def body(h_ref, t_ref, r_ref, o_ref):
        h = h_ref[...]
        t = t_ref[...]
        r = r_ref[...]
        re_h = h[:, :HALF] * scale
        im_h = h[:, HALF:] * scale
        phase = r * inv_phase
        u = phase * phase
        cr = _horner(u, _COS_C)
        sr = _horner(u, _SIN_C) * phase
        re_s = re_h * cr - im_h * sr - t[:, :HALF] * scale
        im_s = re_h * sr + im_h * cr - t[:, HALF:] * scale
        dist = jnp.sqrt(re_s * re_s + im_s * im_s)
        o_ref[...] = GAMMA - jnp.sum(dist, axis=1, keepdims=True)

    return pl.pallas_call(
        body,
        grid=(b // BR,),
        in_specs=[
            pl.BlockSpec((BR, H_DIM), lambda i: (i, 0)),
            pl.BlockSpec((BR, H_DIM), lambda i: (i, 0)),
            pl.BlockSpec((BR, HALF), lambda i: (i, 0)),
        ],
        out_specs=pl.BlockSpec((BR, 1), lambda i: (i, 0)),
        out_shape=jax.ShapeDtypeStruct((b, 1), jnp.float32),
    )(head, tail, rel)


def kernel(embs, sample, w_relation):
    h_idx = sample[0]
    r_idx = sample[1]
    t_idx = sample[2]
    bc = BATCH // NCHUNK
    scores = []
    for i in range(NCHUNK):
        head, tail, rel = _sc_gather(
            embs, w_relation, h_idx, r_idx, t_idx, i * bc, bc)
        scores.append(_tc_score(head, tail, rel))
    return jnp.concatenate(scores, axis=0)


# speedup vs baseline: 1.5297x; 1.3757x over previous
import functools
import math

import jax
import jax.numpy as jnp
from jax import lax
from jax.experimental import pallas as pl
from jax.experimental.pallas import tpu as pltpu
from jax.experimental.pallas import tpu_sc as plsc

H_DIM = 512
HALF = H_DIM // 2
BATCH = 16384
GAMMA = 12.0
EPSILON = 2.0
EMB_RANGE = (GAMMA + EPSILON) / H_DIM
PI = 3.141592653589793

NW = 32
CH = 64
CHUNKS = (2048, 4096, 4096, 4096, 2048)


def _sc_gather(embs, w_relation, h_idx, r_idx, t_idx, chunk_base, bc):
    mesh = plsc.VectorSubcoreMesh(core_axis_name="c", subcore_axis_name="s")
    bpw = bc // NW
    nsub = bpw // CH
    assert nsub in (1, 2)

    @functools.partial(
        pl.kernel,
        mesh=mesh,
        out_type=(
            jax.ShapeDtypeStruct((bc, H_DIM), jnp.float32),
            jax.ShapeDtypeStruct((bc, H_DIM), jnp.float32),
            jax.ShapeDtypeStruct((bc, HALF), jnp.float32),
        ),
        scratch_types=[
            pltpu.VMEM((bpw,), jnp.int32),
            pltpu.VMEM((bpw,), jnp.int32),
            pltpu.VMEM((bpw,), jnp.int32),
            pltpu.VMEM((CH, H_DIM), jnp.float32),
            pltpu.VMEM((CH, H_DIM), jnp.float32),
            pltpu.VMEM((CH, H_DIM), jnp.float32),
            pltpu.VMEM((CH, HALF), jnp.float32),
            pltpu.SemaphoreType.DMA,
            pltpu.SemaphoreType.DMA,
            pltpu.SemaphoreType.DMA,
            pltpu.SemaphoreType.DMA,
            pltpu.SemaphoreType.DMA,
            pltpu.SemaphoreType.DMA,
            pltpu.SemaphoreType.DMA,
            pltpu.SemaphoreType.DMA,
        ],
    )
    def k(embs_h, wrel_h, hidx_h, ridx_h, tidx_h,
          hout, tout, rout,
          hidx_v, ridx_v, tidx_v, b0, b1, b2, rb,
          gb0, gb1, gb2, grb, sb0, sb1, sb2, srb):
        wid = lax.axis_index("s") * 2 + lax.axis_index("c")
        base = wid * bpw
        src = chunk_base + base

        def sub(ref, j):
            return ref.at[pl.ds(j * CH, CH)]

        def out_sub(ref, j):
            return ref.at[pl.ds(base + j * CH, CH)]

        if nsub == 2:
            pltpu.sync_copy(hidx_h.at[pl.ds(src, bpw)], hidx_v)
            gh0 = pltpu.async_copy(embs_h.at[sub(hidx_v, 0)], b0, gb0)
            gh1 = pltpu.async_copy(embs_h.at[sub(hidx_v, 1)], b1, gb1)
            pltpu.sync_copy(tidx_h.at[pl.ds(src, bpw)], tidx_v)
            gt0 = pltpu.async_copy(embs_h.at[sub(tidx_v, 0)], b2, gb2)
            pltpu.sync_copy(ridx_h.at[pl.ds(src, bpw)], ridx_v)
            gr0 = pltpu.async_copy(wrel_h.at[sub(ridx_v, 0)], rb, grb)

            gh0.wait()
            sh0 = pltpu.async_copy(b0, out_sub(hout, 0), sb0)
            gh1.wait()
            sh1 = pltpu.async_copy(b1, out_sub(hout, 1), sb1)
            sh0.wait()
            gt1 = pltpu.async_copy(embs_h.at[sub(tidx_v, 1)], b0, gb0)
            gr0.wait()
            sr0 = pltpu.async_copy(rb, out_sub(rout, 0), srb)
            sr0.wait()
            gr1 = pltpu.async_copy(wrel_h.at[sub(ridx_v, 1)], rb, grb)
            gt0.wait()
            st0 = pltpu.async_copy(b2, out_sub(tout, 0), sb2)
            gt1.wait()
            st1 = pltpu.async_copy(b0, out_sub(tout, 1), sb0)
            gr1.wait()
            sr1 = pltpu.async_copy(rb, out_sub(rout, 1), srb)
            sh1.wait()
            st0.wait()
            st1.wait()
            sr1.wait()
        else:
            pltpu.sync_copy(hidx_h.at[pl.ds(src, bpw)], hidx_v)
            gh0 = pltpu.async_copy(embs_h.at[sub(hidx_v, 0)], b0, gb0)
            pltpu.sync_copy(tidx_h.at[pl.ds(src, bpw)], tidx_v)
            gt0 = pltpu.async_copy(embs_h.at[sub(tidx_v, 0)], b1, gb1)
            pltpu.sync_copy(ridx_h.at[pl.ds(src, bpw)], ridx_v)
            gr0 = pltpu.async_copy(wrel_h.at[sub(ridx_v, 0)], rb, grb)
            gh0.wait()
            sh0 = pltpu.async_copy(b0, out_sub(hout, 0), sb0)
            gt0.wait()
            st0 = pltpu.async_copy(b1, out_sub(tout, 0), sb1)
            gr0.wait()
            sr0 = pltpu.async_copy(rb, out_sub(rout, 0), srb)
            sh0.wait()
            st0.wait()
            sr0.wait()

    return k(embs, w_relation, h_idx, r_idx, t_idx)


_SIN_C = (9.999999992634e-01, -1.666666592737e-01, 8.333321297382e-03,
          -1.984053414314e-04, 2.753585048001e-06, -2.472881380150e-08,
          1.361309747309e-10)
_COS_C = (1.000000000293e+00, -4.999999985941e-01, 4.166666351410e-02,
          -1.388886311125e-03, 2.480055413054e-05, -2.753480385845e-07,
          2.060360183243e-09, -9.722486996111e-12)


def _horner(u, coeffs):
    acc = jnp.full_like(u, coeffs[-1])
    for c in coeffs[-2::-1]:
        acc = acc * u + c
    return acc


def _tc_score(head, tail, rel):
    b = head.shape[0]
    BR = min(1024, b)
    scale = EMB_RANGE / math.sqrt(3.0)
    inv_phase = PI / EMB_RANGE

    def body(h_ref, t_ref, r_ref, o_ref):
        h = h_ref[...]
        t = t_ref[...]
        r = r_ref[...]
        re_h = h[:, :HALF] * scale
        im_h = h[:, HALF:] * scale
        phase = r * inv_phase
        u = phase * phase
        cr = _horner(u, _COS_C)
        sr = _horner(u, _SIN_C) * phase
        re_s = re_h * cr - im_h * sr - t[:, :HALF] * scale
        im_s = re_h * sr + im_h * cr - t[:, HALF:] * scale
        dist = jnp.sqrt(re_s * re_s + im_s * im_s)
        o_ref[...] = GAMMA - jnp.sum(dist, axis=1, keepdims=True)

    return pl.pallas_call(
        body,
        grid=(b // BR,),
        in_specs=[
            pl.BlockSpec((BR, H_DIM), lambda i: (i, 0)),
            pl.BlockSpec((BR, H_DIM), lambda i: (i, 0)),
            pl.BlockSpec((BR, HALF), lambda i: (i, 0)),
        ],
        out_specs=pl.BlockSpec((BR, 1), lambda i: (i, 0)),
        out_shape=jax.ShapeDtypeStruct((b, 1), jnp.float32),
    )(head, tail, rel)


def kernel(embs, sample, w_relation):
    h_idx = sample[0]
    r_idx = sample[1]
    t_idx = sample[2]
    scores = []
    base = 0
    for bc in CHUNKS:
        head, tail, rel = _sc_gather(
            embs, w_relation, h_idx, r_idx, t_idx, base, bc)
        scores.append(_tc_score(head, tail, rel))
        base += bc
    return jnp.concatenate(scores, axis=0)
